# Initial kernel scaffold; baseline (speedup 1.0000x reference)
import jax
import jax.numpy as jnp
from jax.experimental import pallas as pl


def kernel(x, batch, pool_weights, W, b):
    raise NotImplementedError("write your pallas kernel here")


# speedup vs baseline: 5.2418x; 5.2418x over previous
import functools

import jax
import jax.numpy as jnp
from jax import lax
from jax.experimental import pallas as pl
from jax.experimental.pallas import tpu as pltpu
from jax.experimental.pallas import tpu_sc as plsc

N = 100000
HIDDEN = 512
NSEG = 256


_CNT_BLK = 1024
_NPAD = 100352
_NBLKS = _NPAD // _CNT_BLK


def _count_body(ids_ref, lt_ref, cnt_ref):
    i = pl.program_id(0)

    @pl.when(i == 0)
    def _init():
        lt_ref[...] = jnp.zeros_like(lt_ref)
        cnt_ref[...] = jnp.zeros_like(cnt_ref)

    blk = ids_ref[0, 0, :]
    segs = lax.broadcasted_iota(jnp.int32, (NSEG, _CNT_BLK), 0)
    idb = jnp.broadcast_to(blk[None, :], (NSEG, _CNT_BLK))
    lt_ref[...] += jnp.sum((segs > idb).astype(jnp.int32), axis=1)
    cnt_ref[...] += jnp.sum((segs == idb).astype(jnp.int32), axis=1)


_count_call = pl.pallas_call(
    _count_body,
    grid=(_NBLKS,),
    in_specs=[pl.BlockSpec((1, 1, _CNT_BLK), lambda i: (i, 0, 0))],
    out_specs=[pl.BlockSpec((NSEG,), lambda i: (0,)),
               pl.BlockSpec((NSEG,), lambda i: (0,))],
    out_shape=[jax.ShapeDtypeStruct((NSEG,), jnp.int32),
               jax.ShapeDtypeStruct((NSEG,), jnp.int32)],
)


_NC = 2
_NS = 16
_L = 16
_SEGS_PER_G = 16
_COLS_H = 256
_NJ = _COLS_H // _L
_CHUNK = 64
_BUF_ROWS = _CHUNK + 8
_STARTS_PAD = 272


def _lane(vec, k):
    m = lax.broadcasted_iota(jnp.int32, (_L,), 0) == k
    return jnp.sum(jnp.where(m, vec, 0))


def _pool_body(x_hbm, starts_hbm, sum_hbm, max_hbm, sv, buf, st_sum, st_max):
    c = lax.axis_index("c")
    s = lax.axis_index("s")
    wid = s * _NC + c
    g = wid // 2
    h = wid % 2
    col0 = h * _COLS_H

    pltpu.sync_copy(starts_hbm.at[pl.ds(g * _SEGS_PER_G, 32)], sv)
    v0 = sv[pl.ds(0, _L)]
    v1 = sv[pl.ds(_L, _L)]

    for k in range(_SEGS_PER_G):
        r0 = v0[k]
        r1 = v0[k + 1] if k + 1 < _L else v1[0]

        init = ([jnp.zeros((_L,), jnp.float32)] * _NJ,
                [jnp.full((_L,), -jnp.inf, jnp.float32)] * _NJ)
        nchunks = (r1 - r0 + _CHUNK - 1) // _CHUNK

        def chunk_body(ci, accs, r0=r0, r1=r1):
            cstart = r0 + ci * _CHUNK
            off = jnp.minimum((cstart // 8) * 8, N - _BUF_ROWS)
            delta = cstart - off
            pltpu.sync_copy(x_hbm.at[pl.ds(off, _BUF_ROWS),
                                     pl.ds(col0, _COLS_H)],
                            buf)
            nrows = jnp.minimum(_CHUNK, r1 - cstart)

            def row_body(i, accs2, delta=delta):
                sums, maxs = accs2
                r = delta + i
                new_s = []
                new_m = []
                for j in range(_NJ):
                    v = buf[r, pl.ds(j * _L, _L)]
                    new_s.append(sums[j] + v)
                    new_m.append(jnp.maximum(maxs[j], v))
                return (new_s, new_m)

            return lax.fori_loop(0, nrows, row_body, accs)

        sums, maxs = lax.fori_loop(0, nchunks, chunk_body, init)
        for j in range(_NJ):
            st_sum[k, pl.ds(j * _L, _L)] = sums[j]
            st_max[k, pl.ds(j * _L, _L)] = maxs[j]

    pltpu.sync_copy(st_sum,
                    sum_hbm.at[pl.ds(g * _SEGS_PER_G, _SEGS_PER_G),
                               pl.ds(col0, _COLS_H)])
    pltpu.sync_copy(st_max,
                    max_hbm.at[pl.ds(g * _SEGS_PER_G, _SEGS_PER_G),
                               pl.ds(col0, _COLS_H)])


@functools.cache
def _get_pool_call():
    return pl.kernel(
        _pool_body,
        out_type=(jax.ShapeDtypeStruct((NSEG, HIDDEN), jnp.float32),
                  jax.ShapeDtypeStruct((NSEG, HIDDEN), jnp.float32)),
        mesh=plsc.VectorSubcoreMesh(core_axis_name="c", subcore_axis_name="s",
                                    num_cores=_NC, num_subcores=_NS),
        scratch_types=[pltpu.VMEM((32,), jnp.int32),
                       pltpu.VMEM((_BUF_ROWS, _COLS_H), jnp.float32),
                       pltpu.VMEM((_SEGS_PER_G, _COLS_H), jnp.float32),
                       pltpu.VMEM((_SEGS_PER_G, _COLS_H), jnp.float32)],
    )


def _linear_body(sum_ref, max_ref, cnt_ref, pw_ref, W_ref, b_ref, out_ref):
    sums = sum_ref[...]
    maxs = max_ref[...]
    counts = cnt_ref[...].astype(jnp.float32)
    w0 = pw_ref[0]
    w1 = pw_ref[1]
    w2 = pw_ref[2]
    inv = w0 / jnp.maximum(counts, 1.0)
    combined = jnp.concatenate(
        [sums * inv[:, None], w1 * maxs, w2 * sums], axis=1)
    out_ref[...] = lax.dot_general(
        combined, W_ref[...], (((1,), (1,)), ((), ())),
        preferred_element_type=jnp.float32) + b_ref[...][None, :]


_linear_call = pl.pallas_call(
    _linear_body,
    in_specs=[pl.BlockSpec(memory_space=pltpu.VMEM),
              pl.BlockSpec(memory_space=pltpu.VMEM),
              pl.BlockSpec(memory_space=pltpu.VMEM),
              pl.BlockSpec(memory_space=pltpu.SMEM),
              pl.BlockSpec(memory_space=pltpu.VMEM),
              pl.BlockSpec(memory_space=pltpu.VMEM)],
    out_specs=pl.BlockSpec(memory_space=pltpu.VMEM),
    out_shape=jax.ShapeDtypeStruct((NSEG, HIDDEN), jnp.float32),
)


def kernel(x, batch, pool_weights, W, b):
    ids = batch.astype(jnp.int32)
    ids_p = jnp.concatenate(
        [ids, jnp.full((_NPAD - N,), jnp.int32(2**30), jnp.int32)]
    ).reshape(_NBLKS, 1, _CNT_BLK)
    lt, counts = _count_call(ids_p)
    starts = jnp.concatenate(
        [lt, jnp.full((_STARTS_PAD - NSEG,), jnp.int32(N), jnp.int32)])
    sum_pool, max_pool = _get_pool_call()(x, starts)
    return _linear_call(sum_pool, max_pool, counts, pool_weights, W, b)


# speedup vs baseline: 7.2955x; 1.3918x over previous
import functools

import jax
import jax.numpy as jnp
from jax import lax
from jax.experimental import pallas as pl
from jax.experimental.pallas import tpu as pltpu
from jax.experimental.pallas import tpu_sc as plsc

N = 100000
HIDDEN = 512
NSEG = 256


_CNT_BLK = 1024
_NPAD = 100352
_NBLKS = _NPAD // _CNT_BLK


_CNT_ROWS = 14
_CNT_STEPS = _NBLKS // _CNT_ROWS


def _count_body(ids_ref, lt_ref, cnt_ref):
    i = pl.program_id(0)

    @pl.when(i == 0)
    def _init():
        lt_ref[...] = jnp.zeros_like(lt_ref)
        cnt_ref[...] = jnp.zeros_like(cnt_ref)

    segs = lax.broadcasted_iota(jnp.int32, (NSEG, _CNT_BLK), 0)
    lt_acc = jnp.zeros((NSEG,), jnp.int32)
    cnt_acc = jnp.zeros((NSEG,), jnp.int32)
    for rr in range(_CNT_ROWS):
        blk = ids_ref[rr, 0, :]
        idb = jnp.broadcast_to(blk[None, :], (NSEG, _CNT_BLK))
        lt_acc += jnp.sum((segs > idb).astype(jnp.int32), axis=1)
        cnt_acc += jnp.sum((segs == idb).astype(jnp.int32), axis=1)
    lt_ref[...] += lt_acc
    cnt_ref[...] += cnt_acc


_count_call = pl.pallas_call(
    _count_body,
    grid=(_CNT_STEPS,),
    in_specs=[pl.BlockSpec((_CNT_ROWS, 1, _CNT_BLK), lambda i: (i, 0, 0))],
    out_specs=[pl.BlockSpec((NSEG,), lambda i: (0,)),
               pl.BlockSpec((NSEG,), lambda i: (0,))],
    out_shape=[jax.ShapeDtypeStruct((NSEG,), jnp.int32),
               jax.ShapeDtypeStruct((NSEG,), jnp.int32)],
)


_NC = 2
_NS = 16
_L = 16
_SEGS_PER_G = 16
_COLS_H = 256
_NJ = _COLS_H // _L
_CHUNK = 64
_BUF_ROWS = _CHUNK + 8
_STARTS_PAD = 272


def _lane(vec, k):
    m = lax.broadcasted_iota(jnp.int32, (_L,), 0) == k
    return jnp.sum(jnp.where(m, vec, 0))


def _pool_body(x_hbm, starts_hbm, sum_hbm, max_hbm,
               sv, buf0, buf1, st_sum, st_max, sem0, sem1):
    c = lax.axis_index("c")
    s = lax.axis_index("s")
    wid = s * _NC + c
    g = wid // 2
    h = wid % 2
    col0 = h * _COLS_H

    pltpu.sync_copy(starts_hbm.at[pl.ds(g * _SEGS_PER_G, 32)], sv)
    v0 = sv[pl.ds(0, _L)]
    v1 = sv[pl.ds(_L, _L)]

    for k in range(_SEGS_PER_G):
        r0 = v0[k]
        r1 = v0[k + 1] if k + 1 < _L else v1[0]

        init = ([jnp.zeros((_L,), jnp.float32)] * _NJ,
                [jnp.full((_L,), -jnp.inf, jnp.float32)] * _NJ)
        nch = (r1 - r0 + _CHUNK - 1) // _CHUNK

        def _off(ci, r0=r0):
            cstart = r0 + ci * _CHUNK
            return jnp.minimum((cstart // 8) * 8, N - _BUF_ROWS)

        def _copy(ci, dbuf, dsem):
            return pltpu.make_async_copy(
                x_hbm.at[pl.ds(_off(ci), _BUF_ROWS), pl.ds(col0, _COLS_H)],
                dbuf, dsem)

        @pl.when(nch > 0)
        def _prologue():
            _copy(0, buf0, sem0).start()

        def stage(ci, cur, sem_cur, nxt, sem_nxt, accs, r0=r0, r1=r1,
                  nch=nch):
            @pl.when(ci < nch)
            def _arrive():
                _copy(ci, cur, sem_cur).wait()

                @pl.when(ci + 1 < nch)
                def _next():
                    _copy(ci + 1, nxt, sem_nxt).start()

            cstart = r0 + ci * _CHUNK
            delta = cstart - _off(ci)
            nrows = jnp.maximum(0, jnp.minimum(_CHUNK, r1 - cstart))

            def row_body(i, accs2, cur=cur, delta=delta):
                sums, maxs = accs2
                r = delta + i
                new_s = []
                new_m = []
                for j in range(_NJ):
                    v = cur[r, pl.ds(j * _L, _L)]
                    new_s.append(sums[j] + v)
                    new_m.append(jnp.maximum(maxs[j], v))
                return (new_s, new_m)

            return lax.fori_loop(0, nrows, row_body, accs)

        def pair_body(p, accs):
            accs = stage(2 * p, buf0, sem0, buf1, sem1, accs)
            return stage(2 * p + 1, buf1, sem1, buf0, sem0, accs)

        sums, maxs = lax.fori_loop(0, (nch + 1) // 2, pair_body, init)
        for j in range(_NJ):
            st_sum[k, pl.ds(j * _L, _L)] = sums[j]
            st_max[k, pl.ds(j * _L, _L)] = maxs[j]

    pltpu.sync_copy(st_sum,
                    sum_hbm.at[pl.ds(g * _SEGS_PER_G, _SEGS_PER_G),
                               pl.ds(col0, _COLS_H)])
    pltpu.sync_copy(st_max,
                    max_hbm.at[pl.ds(g * _SEGS_PER_G, _SEGS_PER_G),
                               pl.ds(col0, _COLS_H)])


@functools.cache
def _get_pool_call():
    return pl.kernel(
        _pool_body,
        out_type=(jax.ShapeDtypeStruct((NSEG, HIDDEN), jnp.float32),
                  jax.ShapeDtypeStruct((NSEG, HIDDEN), jnp.float32)),
        mesh=plsc.VectorSubcoreMesh(core_axis_name="c", subcore_axis_name="s",
                                    num_cores=_NC, num_subcores=_NS),
        scratch_types=[pltpu.VMEM((32,), jnp.int32),
                       pltpu.VMEM((_BUF_ROWS, _COLS_H), jnp.float32),
                       pltpu.VMEM((_BUF_ROWS, _COLS_H), jnp.float32),
                       pltpu.VMEM((_SEGS_PER_G, _COLS_H), jnp.float32),
                       pltpu.VMEM((_SEGS_PER_G, _COLS_H), jnp.float32),
                       pltpu.SemaphoreType.DMA,
                       pltpu.SemaphoreType.DMA],
    )


def _linear_body(sum_ref, max_ref, cnt_ref, pw_ref, W_ref, b_ref, out_ref):
    sums = sum_ref[...]
    maxs = max_ref[...]
    counts = cnt_ref[...].astype(jnp.float32)
    w0 = pw_ref[0]
    w1 = pw_ref[1]
    w2 = pw_ref[2]
    inv = w0 / jnp.maximum(counts, 1.0)
    combined = jnp.concatenate(
        [sums * inv[:, None], w1 * maxs, w2 * sums], axis=1)
    out_ref[...] = lax.dot_general(
        combined, W_ref[...], (((1,), (1,)), ((), ())),
        preferred_element_type=jnp.float32) + b_ref[...][None, :]


_linear_call = pl.pallas_call(
    _linear_body,
    in_specs=[pl.BlockSpec(memory_space=pltpu.VMEM),
              pl.BlockSpec(memory_space=pltpu.VMEM),
              pl.BlockSpec(memory_space=pltpu.VMEM),
              pl.BlockSpec(memory_space=pltpu.SMEM),
              pl.BlockSpec(memory_space=pltpu.VMEM),
              pl.BlockSpec(memory_space=pltpu.VMEM)],
    out_specs=pl.BlockSpec(memory_space=pltpu.VMEM),
    out_shape=jax.ShapeDtypeStruct((NSEG, HIDDEN), jnp.float32),
)


def kernel(x, batch, pool_weights, W, b):
    ids = batch.astype(jnp.int32)
    ids_p = jnp.concatenate(
        [ids, jnp.full((_NPAD - N,), jnp.int32(2**30), jnp.int32)]
    ).reshape(_NBLKS, 1, _CNT_BLK)
    lt, counts = _count_call(ids_p)
    starts = jnp.concatenate(
        [lt, jnp.full((_STARTS_PAD - NSEG,), jnp.int32(N), jnp.int32)])
    sum_pool, max_pool = _get_pool_call()(x, starts)
    return _linear_call(sum_pool, max_pool, counts, pool_weights, W, b)
